# Initial kernel scaffold; baseline (speedup 1.0000x reference)
import jax
import jax.numpy as jnp
from jax.experimental import pallas as pl


def kernel(x, W_ih, W_hh, b_ih, b_hh, We1, be1, Wn1, bn1, We2, be2, Wn2, bn2, Wd, bd):
    raise NotImplementedError("write your pallas kernel here")


# speedup vs baseline: 18.4249x; 18.4249x over previous
import functools

import jax
import jax.numpy as jnp
from jax.experimental import pallas as pl
from jax.experimental.pallas import tpu as pltpu

NUM_NODES = 64
K = 8
HIDDEN = 128
OUT_SEQ = 12


def _adj_body(x_ref, adj_ref):
    xb = x_ref[0]
    n = xb.shape[0]
    row = jax.lax.broadcasted_iota(jnp.int32, (n, n), 0)
    col = jax.lax.broadcasted_iota(jnp.int32, (n, n), 1)
    eye = row == col
    nrm = jnp.sqrt(jnp.sum(xb * xb, axis=1, keepdims=True))
    xn = (xb / (nrm + 1e-12)).astype(jnp.bfloat16)
    sim = jax.lax.dot_general(
        xn, xn, (((1,), (1,)), ((), ())),
        preferred_element_type=jnp.float32)

    s = sim
    keep = jnp.zeros((n, n), jnp.float32)
    for _ in range(K):
        mx = jnp.max(s, axis=1, keepdims=True)
        sel = s == mx
        first = jnp.min(jnp.where(sel, col, n), axis=1, keepdims=True)
        mk = col == first
        keep = jnp.where(mk, 1.0, keep)
        s = jnp.where(mk, -jnp.inf, s)

    eyef = jnp.where(eye, 1.0, 0.0)
    keep_t = jax.lax.dot_general(
        eyef, keep, (((1,), (1,)), ((), ())),
        preferred_element_type=jnp.float32,
        precision=jax.lax.Precision.HIGHEST)
    adj = sim * (keep + keep_t) * 0.5
    adj = jnp.where(eye, 1.0, adj)
    adj_ref[0] = adj


def _build_adj(x3):
    b, n, f = x3.shape
    return pl.pallas_call(
        _adj_body,
        grid=(b,),
        in_specs=[pl.BlockSpec((1, n, f), lambda i: (i, 0, 0))],
        out_specs=pl.BlockSpec((1, n, n), lambda i: (i, 0, 0)),
        out_shape=jax.ShapeDtypeStruct((b, n, n), jnp.float32),
    )(x3)


def _gru_body(x_ref, wih_ref, whh_ref, bih_ref, bhh_ref, out_ref, h_ref,
              *, t_total):
    t = pl.program_id(0)

    @pl.when(t == 0)
    def _():
        h_ref[...] = jnp.zeros_like(h_ref)

    h = HIDDEN
    xt = x_ref[0]
    xp = jnp.dot(xt, wih_ref[...], preferred_element_type=jnp.float32,
                 precision=jax.lax.Precision.HIGHEST)
    xp = xp + bih_ref[...]
    hprev = h_ref[...]
    gh = jnp.dot(hprev, whh_ref[...], preferred_element_type=jnp.float32,
                 precision=jax.lax.Precision.HIGHEST)
    gh = gh + bhh_ref[...]
    r = jax.nn.sigmoid(xp[:, :h] + gh[:, :h])
    z = jax.nn.sigmoid(xp[:, h:2 * h] + gh[:, h:2 * h])
    nn = jnp.tanh(xp[:, 2 * h:] + r * gh[:, 2 * h:])
    hnew = (1.0 - z) * nn + z * hprev
    h_ref[...] = hnew

    @pl.when(t >= t_total - OUT_SEQ)
    def _():
        out_ref[pl.ds(t - (t_total - OUT_SEQ), 1)] = hnew[None]


def _run_gru(xt_first, wih_t, whh_t, bih, bhh):
    t_total, bn, d = xt_first.shape
    return pl.pallas_call(
        functools.partial(_gru_body, t_total=t_total),
        grid=(t_total,),
        in_specs=[
            pl.BlockSpec((1, bn, d), lambda t: (t, 0, 0)),
            pl.BlockSpec((d, 3 * HIDDEN), lambda t: (0, 0)),
            pl.BlockSpec((HIDDEN, 3 * HIDDEN), lambda t: (0, 0)),
            pl.BlockSpec((1, 3 * HIDDEN), lambda t: (0, 0)),
            pl.BlockSpec((1, 3 * HIDDEN), lambda t: (0, 0)),
        ],
        out_specs=pl.BlockSpec((OUT_SEQ, bn, HIDDEN), lambda t: (0, 0, 0)),
        out_shape=jax.ShapeDtypeStruct((OUT_SEQ, bn, HIDDEN), jnp.float32),
        scratch_shapes=[pltpu.VMEM((bn, HIDDEN), jnp.float32)],
    )(xt_first, wih_t, whh_t, bih, bhh)


def _gine_layer_local(xl, a, maskf, we_ref, be_ref, wn_ref, bn_ref):
    n = xl.shape[0]
    we = we_ref[...][None]
    be = be_ref[...][None]
    m = jnp.zeros((n, HIDDEN), jnp.float32)
    cj = 8
    for j0 in range(0, n, cj):
        xj = xl[j0:j0 + cj, :]
        aj = a[j0:j0 + cj, :]
        mj = maskf[j0:j0 + cj, :]
        ej = xj[:, None, :] + aj[:, :, None] * we + be
        msg = jnp.maximum(ej, 0.0) * mj[:, :, None]
        m = m + jnp.sum(msg, axis=0)
    out = jnp.dot(xl + m, wn_ref[...], preferred_element_type=jnp.float32,
                  precision=jax.lax.Precision.HIGHEST)
    return out + bn_ref[...]


def _gine_body(h_ref, adj_ref, we1_ref, be1_ref, wn1_ref, bn1_ref,
               we2_ref, be2_ref, wn2_ref, bn2_ref, wd_ref, bd_ref, out_ref):
    xi = h_ref[0, 0]
    a = adj_ref[0]
    maskf = jnp.where(a != 0.0, 1.0, 0.0)
    x1 = jax.nn.relu(
        _gine_layer_local(xi, a, maskf, we1_ref, be1_ref, wn1_ref, bn1_ref))
    x2 = jax.nn.relu(
        _gine_layer_local(x1, a, maskf, we2_ref, be2_ref, wn2_ref, bn2_ref))
    s = jnp.sum(x2 * wd_ref[...], axis=1, keepdims=True) + bd_ref[0, 0]
    out_ref[0, 0] = jnp.broadcast_to(s, x2.shape)


def _run_gine(hseq, adjb, we1, be1, wn1_t, bn1, we2, be2, wn2_t, bn2, wd, bd,
              t_total):
    b, t2, n, h = hseq.shape
    vec = pl.BlockSpec((1, h), lambda bi, ti: (0, 0))
    mat = pl.BlockSpec((h, h), lambda bi, ti: (0, 0))
    return pl.pallas_call(
        _gine_body,
        grid=(b, t2),
        in_specs=[
            pl.BlockSpec((1, 1, n, h), lambda bi, ti: (bi, ti, 0, 0)),
            pl.BlockSpec(
                (1, n, n),
                lambda bi, ti: ((bi * t_total + (t_total - t2) + ti) % b, 0, 0)),
            vec, vec, mat, vec,
            vec, vec, mat, vec,
            vec, pl.BlockSpec((1, 1), lambda bi, ti: (0, 0)),
        ],
        out_specs=pl.BlockSpec((1, 1, n, h), lambda bi, ti: (bi, ti, 0, 0)),
        out_shape=jax.ShapeDtypeStruct((b, t2, n, h), jnp.float32),
    )(hseq, adjb, we1, be1, wn1_t, bn1, we2, be2, wn2_t, bn2, wd, bd)


def kernel(x, W_ih, W_hh, b_ih, b_hh, We1, be1, Wn1, bn1, We2, be2, Wn2, bn2,
           Wd, bd):
    bn, t, d = x.shape
    batch = bn // NUM_NODES

    adjb = _build_adj(x.reshape(batch, NUM_NODES, t * d))

    xt_first = jnp.swapaxes(x, 0, 1)
    hs = _run_gru(xt_first, W_ih.T, W_hh.T, b_ih[None], b_hh[None])
    hseq = hs.reshape(OUT_SEQ, batch, NUM_NODES, HIDDEN).transpose(1, 0, 2, 3)

    outf = _run_gine(hseq, adjb,
                     We1[None], be1[None], Wn1.T, bn1[None],
                     We2[None], be2[None], Wn2.T, bn2[None],
                     Wd, bd.reshape(1, 1), t)
    out = outf[..., :1].transpose(0, 2, 1, 3).reshape(bn, OUT_SEQ, 1)
    return out


# speedup vs baseline: 22.0190x; 1.1951x over previous
import functools

import jax
import jax.numpy as jnp
from jax.experimental import pallas as pl
from jax.experimental.pallas import tpu as pltpu

NUM_NODES = 64
K = 8
HIDDEN = 128
OUT_SEQ = 12


def _adj_body(x_ref, adj_ref):
    xb = x_ref[0]
    n = xb.shape[0]
    row = jax.lax.broadcasted_iota(jnp.int32, (n, n), 0)
    col = jax.lax.broadcasted_iota(jnp.int32, (n, n), 1)
    eye = row == col
    nrm = jnp.sqrt(jnp.sum(xb * xb, axis=1, keepdims=True))
    xn = (xb / (nrm + 1e-12)).astype(jnp.bfloat16)
    sim = jax.lax.dot_general(
        xn, xn, (((1,), (1,)), ((), ())),
        preferred_element_type=jnp.float32)

    s = sim
    keep = jnp.zeros((n, n), jnp.float32)
    for _ in range(K):
        mx = jnp.max(s, axis=1, keepdims=True)
        sel = s == mx
        first = jnp.min(jnp.where(sel, col, n), axis=1, keepdims=True)
        mk = col == first
        keep = jnp.where(mk, 1.0, keep)
        s = jnp.where(mk, -jnp.inf, s)

    eyef = jnp.where(eye, 1.0, 0.0)
    keep_t = jax.lax.dot_general(
        eyef, keep, (((1,), (1,)), ((), ())),
        preferred_element_type=jnp.float32,
        precision=jax.lax.Precision.HIGHEST)
    adj = sim * (keep + keep_t) * 0.5
    adj = jnp.where(eye, 1.0, adj)
    adj_ref[0] = adj


def _build_adj(x3):
    b, n, f = x3.shape
    return pl.pallas_call(
        _adj_body,
        grid=(b,),
        in_specs=[pl.BlockSpec((1, n, f), lambda i: (i, 0, 0))],
        out_specs=pl.BlockSpec((1, n, n), lambda i: (i, 0, 0)),
        out_shape=jax.ShapeDtypeStruct((b, n, n), jnp.float32),
    )(x3)


def _gru_body(x_ref, wih_ref, whh_ref, bih_ref, bhh_ref, out_ref, h_ref,
              *, t_total):
    t = pl.program_id(0)

    @pl.when(t == 0)
    def _():
        h_ref[...] = jnp.zeros_like(h_ref)

    h = HIDDEN
    xt = x_ref[0].astype(jnp.bfloat16)
    xp = jnp.dot(xt, wih_ref[...].astype(jnp.bfloat16),
                 preferred_element_type=jnp.float32)
    xp = xp + bih_ref[...]
    hprev = h_ref[...]
    gh = jnp.dot(hprev.astype(jnp.bfloat16), whh_ref[...].astype(jnp.bfloat16),
                 preferred_element_type=jnp.float32)
    gh = gh + bhh_ref[...]
    r = jax.nn.sigmoid(xp[:, :h] + gh[:, :h])
    z = jax.nn.sigmoid(xp[:, h:2 * h] + gh[:, h:2 * h])
    nn = jnp.tanh(xp[:, 2 * h:] + r * gh[:, 2 * h:])
    hnew = (1.0 - z) * nn + z * hprev
    h_ref[...] = hnew

    @pl.when(t >= t_total - OUT_SEQ)
    def _():
        out_ref[pl.ds(t - (t_total - OUT_SEQ), 1)] = hnew[None]


def _run_gru(xt_first, wih_t, whh_t, bih, bhh):
    t_total, bn, d = xt_first.shape
    return pl.pallas_call(
        functools.partial(_gru_body, t_total=t_total),
        grid=(t_total,),
        in_specs=[
            pl.BlockSpec((1, bn, d), lambda t: (t, 0, 0)),
            pl.BlockSpec((d, 3 * HIDDEN), lambda t: (0, 0)),
            pl.BlockSpec((HIDDEN, 3 * HIDDEN), lambda t: (0, 0)),
            pl.BlockSpec((1, 3 * HIDDEN), lambda t: (0, 0)),
            pl.BlockSpec((1, 3 * HIDDEN), lambda t: (0, 0)),
        ],
        out_specs=pl.BlockSpec((OUT_SEQ, bn, HIDDEN), lambda t: (0, 0, 0)),
        out_shape=jax.ShapeDtypeStruct((OUT_SEQ, bn, HIDDEN), jnp.float32),
        scratch_shapes=[pltpu.VMEM((bn, HIDDEN), jnp.float32)],
    )(xt_first, wih_t, whh_t, bih, bhh)


def _gine_layer_local(xl, a, maskf, we_ref, be_ref, wn_ref, bn_ref):
    n = xl.shape[0]
    we = we_ref[...][None]
    be = be_ref[...][None]
    m = jnp.zeros((n, HIDDEN), jnp.float32)
    cj = 8
    for j0 in range(0, n, cj):
        xj = xl[j0:j0 + cj, :]
        aj = a[j0:j0 + cj, :]
        mj = maskf[j0:j0 + cj, :]
        ej = xj[:, None, :] + aj[:, :, None] * we + be
        msg = jnp.maximum(ej, 0.0) * mj[:, :, None]
        m = m + jnp.sum(msg, axis=0)
    out = jnp.dot((xl + m).astype(jnp.bfloat16),
                  wn_ref[...].astype(jnp.bfloat16),
                  preferred_element_type=jnp.float32)
    return out + bn_ref[...]


def _gine_body(h_ref, adj_ref, we1_ref, be1_ref, wn1_ref, bn1_ref,
               we2_ref, be2_ref, wn2_ref, bn2_ref, wd_ref, bd_ref, out_ref):
    xi = h_ref[0, 0]
    a = adj_ref[0]
    maskf = jnp.where(a != 0.0, 1.0, 0.0)
    x1 = jax.nn.relu(
        _gine_layer_local(xi, a, maskf, we1_ref, be1_ref, wn1_ref, bn1_ref))
    x2 = jax.nn.relu(
        _gine_layer_local(x1, a, maskf, we2_ref, be2_ref, wn2_ref, bn2_ref))
    prod = (x2.astype(jnp.bfloat16).astype(jnp.float32)
            * wd_ref[...].astype(jnp.bfloat16).astype(jnp.float32))
    s = jnp.sum(prod, axis=1, keepdims=True) + bd_ref[0, 0]
    out_ref[0, 0] = jnp.broadcast_to(s, x2.shape)


def _run_gine(hseq, adjb, we1, be1, wn1_t, bn1, we2, be2, wn2_t, bn2, wd, bd,
              t_total):
    b, t2, n, h = hseq.shape
    vec = pl.BlockSpec((1, h), lambda bi, ti: (0, 0))
    mat = pl.BlockSpec((h, h), lambda bi, ti: (0, 0))
    return pl.pallas_call(
        _gine_body,
        grid=(b, t2),
        in_specs=[
            pl.BlockSpec((1, 1, n, h), lambda bi, ti: (bi, ti, 0, 0)),
            pl.BlockSpec(
                (1, n, n),
                lambda bi, ti: ((bi * t_total + (t_total - t2) + ti) % b, 0, 0)),
            vec, vec, mat, vec,
            vec, vec, mat, vec,
            vec, pl.BlockSpec((1, 1), lambda bi, ti: (0, 0)),
        ],
        out_specs=pl.BlockSpec((1, 1, n, h), lambda bi, ti: (bi, ti, 0, 0)),
        out_shape=jax.ShapeDtypeStruct((b, t2, n, h), jnp.float32),
    )(hseq, adjb, we1, be1, wn1_t, bn1, we2, be2, wn2_t, bn2, wd, bd)


def kernel(x, W_ih, W_hh, b_ih, b_hh, We1, be1, Wn1, bn1, We2, be2, Wn2, bn2,
           Wd, bd):
    bn, t, d = x.shape
    batch = bn // NUM_NODES

    adjb = _build_adj(x.reshape(batch, NUM_NODES, t * d))

    xt_first = jnp.swapaxes(x, 0, 1)
    hs = _run_gru(xt_first, W_ih.T, W_hh.T, b_ih[None], b_hh[None])
    hseq = hs.reshape(OUT_SEQ, batch, NUM_NODES, HIDDEN).transpose(1, 0, 2, 3)

    outf = _run_gine(hseq, adjb,
                     We1[None], be1[None], Wn1.T, bn1[None],
                     We2[None], be2[None], Wn2.T, bn2[None],
                     Wd, bd.reshape(1, 1), t)
    out = outf[..., :1].transpose(0, 2, 1, 3).reshape(bn, OUT_SEQ, 1)
    return out


# speedup vs baseline: 30.2156x; 1.3723x over previous
import functools

import jax
import jax.numpy as jnp
from jax.experimental import pallas as pl
from jax.experimental.pallas import tpu as pltpu

NUM_NODES = 64
K = 8
HIDDEN = 128
OUT_SEQ = 12


def _adj_body(x_ref, adj_ref):
    xb = x_ref[0]
    n = xb.shape[0]
    row = jax.lax.broadcasted_iota(jnp.int32, (n, n), 0)
    col = jax.lax.broadcasted_iota(jnp.int32, (n, n), 1)
    eye = row == col
    nrm = jnp.sqrt(jnp.sum(xb * xb, axis=1, keepdims=True))
    xn = (xb / (nrm + 1e-12)).astype(jnp.bfloat16)
    sim = jax.lax.dot_general(
        xn, xn, (((1,), (1,)), ((), ())),
        preferred_element_type=jnp.float32)

    s = sim
    keep = jnp.zeros((n, n), jnp.float32)
    for _ in range(K):
        mx = jnp.max(s, axis=1, keepdims=True)
        sel = s == mx
        first = jnp.min(jnp.where(sel, col, n), axis=1, keepdims=True)
        mk = col == first
        keep = jnp.where(mk, 1.0, keep)
        s = jnp.where(mk, -jnp.inf, s)

    eyef = jnp.where(eye, 1.0, 0.0)
    keep_t = jax.lax.dot_general(
        eyef, keep, (((1,), (1,)), ((), ())),
        preferred_element_type=jnp.float32,
        precision=jax.lax.Precision.HIGHEST)
    adj = sim * (keep + keep_t) * 0.5
    adj = jnp.where(eye, 1.0, adj)
    adj_ref[0] = adj


def _build_adj(x3):
    b, n, f = x3.shape
    return pl.pallas_call(
        _adj_body,
        grid=(b,),
        in_specs=[pl.BlockSpec((1, n, f), lambda i: (i, 0, 0))],
        out_specs=pl.BlockSpec((1, n, n), lambda i: (i, 0, 0)),
        out_shape=jax.ShapeDtypeStruct((b, n, n), jnp.float32),
    )(x3)


GRU_CHUNK = 8


def _gru_body(x_ref, wih_ref, whh_ref, bih_ref, bhh_ref, out_ref, h_ref,
              *, t_total):
    c = pl.program_id(0)

    @pl.when(c == 0)
    def _():
        h_ref[...] = jnp.zeros_like(h_ref)

    h = HIDDEN
    wihb = wih_ref[...].astype(jnp.bfloat16)
    whhb = whh_ref[...].astype(jnp.bfloat16)
    bih = bih_ref[...]
    bhh = bhh_ref[...]
    xc = x_ref[...]
    hprev = h_ref[...]
    for j in range(GRU_CHUNK):
        xt = xc[:, j, :].astype(jnp.bfloat16)
        xp = jnp.dot(xt, wihb, preferred_element_type=jnp.float32) + bih
        gh = jnp.dot(hprev.astype(jnp.bfloat16), whhb,
                     preferred_element_type=jnp.float32) + bhh
        r = jax.nn.sigmoid(xp[:, :h] + gh[:, :h])
        z = jax.nn.sigmoid(xp[:, h:2 * h] + gh[:, h:2 * h])
        nn = jnp.tanh(xp[:, 2 * h:] + r * gh[:, 2 * h:])
        hprev = (1.0 - z) * nn + z * hprev
        t_abs = c * GRU_CHUNK + j

        @pl.when(t_abs >= t_total - OUT_SEQ)
        def _(hnew=hprev, t_abs=t_abs):
            out_ref[pl.ds(t_abs - (t_total - OUT_SEQ), 1)] = hnew[None]

    h_ref[...] = hprev


def _run_gru(x, wih_t, whh_t, bih, bhh):
    bn, t_total, d = x.shape
    return pl.pallas_call(
        functools.partial(_gru_body, t_total=t_total),
        grid=(t_total // GRU_CHUNK,),
        in_specs=[
            pl.BlockSpec((bn, GRU_CHUNK, d), lambda c: (0, c, 0)),
            pl.BlockSpec((d, 3 * HIDDEN), lambda c: (0, 0)),
            pl.BlockSpec((HIDDEN, 3 * HIDDEN), lambda c: (0, 0)),
            pl.BlockSpec((1, 3 * HIDDEN), lambda c: (0, 0)),
            pl.BlockSpec((1, 3 * HIDDEN), lambda c: (0, 0)),
        ],
        out_specs=pl.BlockSpec((OUT_SEQ, bn, HIDDEN), lambda c: (0, 0, 0)),
        out_shape=jax.ShapeDtypeStruct((OUT_SEQ, bn, HIDDEN), jnp.float32),
        scratch_shapes=[pltpu.VMEM((bn, HIDDEN), jnp.float32)],
    )(x, wih_t, whh_t, bih, bhh)


def _gine_layer_local(xl, a, maskf, we_ref, be_ref, wn_ref, bn_ref):
    n = xl.shape[0]
    we = we_ref[...]
    be = be_ref[...]
    m = jnp.zeros((n, HIDDEN), jnp.float32)
    for j in range(n):
        acol = a[:, j:j + 1]
        mcol = maskf[:, j:j + 1]
        ej = (xl[j:j + 1, :] + acol * we) + be
        m = m + jnp.maximum(ej, 0.0) * mcol
    out = jnp.dot((xl + m).astype(jnp.bfloat16),
                  wn_ref[...].astype(jnp.bfloat16),
                  preferred_element_type=jnp.float32)
    return out + bn_ref[...]


def _gine_body(h_ref, adj_ref, we1_ref, be1_ref, wn1_ref, bn1_ref,
               we2_ref, be2_ref, wn2_ref, bn2_ref, wd_ref, bd_ref, out_ref):
    xi = h_ref[0, 0]
    a = adj_ref[0]
    maskf = jnp.where(a != 0.0, 1.0, 0.0)
    x1 = jax.nn.relu(
        _gine_layer_local(xi, a, maskf, we1_ref, be1_ref, wn1_ref, bn1_ref))
    x2 = jax.nn.relu(
        _gine_layer_local(x1, a, maskf, we2_ref, be2_ref, wn2_ref, bn2_ref))
    prod = (x2.astype(jnp.bfloat16).astype(jnp.float32)
            * wd_ref[...].astype(jnp.bfloat16).astype(jnp.float32))
    s = jnp.sum(prod, axis=1, keepdims=True) + bd_ref[0, 0]
    out_ref[0, 0] = jnp.broadcast_to(s, x2.shape)


def _run_gine(hseq, adjb, we1, be1, wn1_t, bn1, we2, be2, wn2_t, bn2, wd, bd,
              t_total):
    b, t2, n, h = hseq.shape
    vec = pl.BlockSpec((1, h), lambda bi, ti: (0, 0))
    mat = pl.BlockSpec((h, h), lambda bi, ti: (0, 0))
    return pl.pallas_call(
        _gine_body,
        grid=(b, t2),
        in_specs=[
            pl.BlockSpec((1, 1, n, h), lambda bi, ti: (bi, ti, 0, 0)),
            pl.BlockSpec(
                (1, n, n),
                lambda bi, ti: ((bi * t_total + (t_total - t2) + ti) % b, 0, 0)),
            vec, vec, mat, vec,
            vec, vec, mat, vec,
            vec, pl.BlockSpec((1, 1), lambda bi, ti: (0, 0)),
        ],
        out_specs=pl.BlockSpec((1, 1, n, h), lambda bi, ti: (bi, ti, 0, 0)),
        out_shape=jax.ShapeDtypeStruct((b, t2, n, h), jnp.float32),
    )(hseq, adjb, we1, be1, wn1_t, bn1, we2, be2, wn2_t, bn2, wd, bd)


def kernel(x, W_ih, W_hh, b_ih, b_hh, We1, be1, Wn1, bn1, We2, be2, Wn2, bn2,
           Wd, bd):
    bn, t, d = x.shape
    batch = bn // NUM_NODES

    adjb = _build_adj(x.reshape(batch, NUM_NODES, t * d))

    hs = _run_gru(x, W_ih.T, W_hh.T, b_ih[None], b_hh[None])
    hseq = hs.reshape(OUT_SEQ, batch, NUM_NODES, HIDDEN).transpose(1, 0, 2, 3)

    outf = _run_gine(hseq, adjb,
                     We1[None], be1[None], Wn1.T, bn1[None],
                     We2[None], be2[None], Wn2.T, bn2[None],
                     Wd, bd.reshape(1, 1), t)
    out = outf[..., :1].transpose(0, 2, 1, 3).reshape(bn, OUT_SEQ, 1)
    return out
